# baseline (speedup 1.0000x reference)
import functools

import jax
import jax.numpy as jnp
from jax import lax
from jax.experimental import pallas as pl
from jax.experimental.pallas import tpu as pltpu
from jax.experimental.pallas import tpu_sc as plsc

D = 256
K = 64
NW = 32
CH = 64
L = 16


def _prep_kernel(xkT_ref, dpT_ref, sc_ref, sh_ref, aT_ref, bT_ref):
    f32 = jnp.float32
    xkT = xkT_ref[...]
    dpT = dpT_ref[...]
    row = lax.broadcasted_iota(jnp.int32, (K, K), 0)
    col = lax.broadcasted_iota(jnp.int32, (K, K), 1)
    rmask = lax.broadcasted_iota(jnp.int32, (K, D), 0)

    m1 = (col == row + 1).astype(f32) - (col == row).astype(f32)
    dxT = jnp.dot(m1, xkT, preferred_element_type=f32)
    dxT = jnp.where(rmask == K - 1, 0.0, dxT)

    sT = jax.nn.softplus(dpT) + 1e-4
    avg = jnp.sum(sT * dxT, axis=0, keepdims=True) / (
        jnp.sum(dxT, axis=0, keepdims=True) + 1e-8)
    sT = sT / (avg + 1e-8)

    tm = (col < row).astype(f32)
    ykT = jnp.dot(tm, sT * dxT, preferred_element_type=f32)

    scale = jax.nn.softplus(sc_ref[...]) + 1e-3
    shiftv = sh_ref[...]
    aT_ref[...] = sT * scale
    bT_ref[...] = (ykT - sT * xkT) * scale + shiftv


def _sc_spline(x_hbm, xkf_hbm, af_hbm, bf_hbm, out_hbm,
               xkv, av, bv, xin, yout):
    rows_per_w = x_hbm.shape[0] // (NW * D)
    nchunk = rows_per_w // CH
    wid = lax.axis_index("s") * 2 + lax.axis_index("c")
    base = wid * rows_per_w * D

    pltpu.sync_copy(xkf_hbm, xkv)
    pltpu.sync_copy(af_hbm, av)
    pltpu.sync_copy(bf_hbm, bv)

    iota = lax.iota(jnp.int32, L)
    ibs = [iota * K + v * L * K for v in range(D // L)]

    def chunk_body(c, carry):
        e0 = base + c * CH * D
        pltpu.sync_copy(x_hbm.at[pl.ds(e0, CH * D)], xin)

        def row_body(r, rc):
            off = r * D
            for v in range(D // L):
                xv = xin[pl.ds(off + v * L, L)]
                ib = ibs[v]
                labs = ib
                for s in (32, 16, 8, 4, 2, 1):
                    g = plsc.load_gather(xkv, [labs + (s - 1)])
                    labs = jnp.where(g < xv, labs + s, labs)
                ia = jnp.minimum(jnp.maximum(labs - 1, ib), ib + (K - 2))
                a = plsc.load_gather(av, [ia])
                b = plsc.load_gather(bv, [ia])
                yout[pl.ds(off + v * L, L)] = a * xv + b
            return rc

        lax.fori_loop(0, CH, row_body, 0)
        pltpu.sync_copy(yout, out_hbm.at[pl.ds(e0, CH * D)])
        return carry

    lax.fori_loop(0, nchunk, chunk_body, 0)


def kernel(x, xk, delta_raw, scale_raw, shift):
    f32 = jnp.float32
    n = x.shape[0]
    xkT = xk.T.astype(f32)
    dpT = jnp.pad(delta_raw, ((0, 0), (0, 1))).T.astype(f32)
    sc = scale_raw[None, :].astype(f32)
    sh = shift[None, :].astype(f32)

    aT, bT = pl.pallas_call(
        _prep_kernel,
        out_shape=[jax.ShapeDtypeStruct((K, D), f32)] * 2,
    )(xkT, dpT, sc, sh)

    xkf = xk.astype(f32).reshape(-1)
    af = aT.T.reshape(-1)
    bf = bT.T.reshape(-1)
    xf = x.reshape(-1)

    mesh = plsc.VectorSubcoreMesh(core_axis_name="c", subcore_axis_name="s")
    run = functools.partial(
        pl.kernel,
        mesh=mesh,
        compiler_params=pltpu.CompilerParams(needs_layout_passes=False),
        out_type=jax.ShapeDtypeStruct((n * D,), f32),
        scratch_types=[
            pltpu.VMEM((D * K,), f32),
            pltpu.VMEM((D * K,), f32),
            pltpu.VMEM((D * K,), f32),
            pltpu.VMEM((CH * D,), f32),
            pltpu.VMEM((CH * D,), f32),
        ],
    )(_sc_spline)
    out = run(xf, xkf, af, bf)
    return out.reshape(n, D)


# speedup vs baseline: 1.9880x; 1.9880x over previous
import functools

import jax
import jax.numpy as jnp
from jax import lax
from jax.experimental import pallas as pl
from jax.experimental.pallas import tpu as pltpu
from jax.experimental.pallas import tpu_sc as plsc

D = 256
K = 64
NW = 32
CH = 64
L = 16


def _prep_kernel(xkT_ref, dpT_ref, sc_ref, sh_ref, aT_ref, bT_ref):
    f32 = jnp.float32
    xkT = xkT_ref[...]
    dpT = dpT_ref[...]
    row = lax.broadcasted_iota(jnp.int32, (K, K), 0)
    col = lax.broadcasted_iota(jnp.int32, (K, K), 1)
    rmask = lax.broadcasted_iota(jnp.int32, (K, D), 0)

    m1 = (col == row + 1).astype(f32) - (col == row).astype(f32)
    dxT = jnp.dot(m1, xkT, preferred_element_type=f32)
    dxT = jnp.where(rmask == K - 1, 0.0, dxT)

    sT = jax.nn.softplus(dpT) + 1e-4
    avg = jnp.sum(sT * dxT, axis=0, keepdims=True) / (
        jnp.sum(dxT, axis=0, keepdims=True) + 1e-8)
    sT = sT / (avg + 1e-8)

    tm = (col < row).astype(f32)
    ykT = jnp.dot(tm, sT * dxT, preferred_element_type=f32)

    scale = jax.nn.softplus(sc_ref[...]) + 1e-3
    shiftv = sh_ref[...]
    aT_ref[...] = sT * scale
    bT_ref[...] = (ykT - sT * xkT) * scale + shiftv


def _sc_spline(x_hbm, xkf_hbm, af_hbm, bf_hbm, out_hbm,
               xkv, av, bv, xin, yout):
    rows_per_w = x_hbm.shape[0] // (NW * D)
    nchunk = rows_per_w // CH
    wid = lax.axis_index("s") * 2 + lax.axis_index("c")
    base = wid * rows_per_w * D

    pltpu.sync_copy(xkf_hbm, xkv)
    pltpu.sync_copy(af_hbm, av)
    pltpu.sync_copy(bf_hbm, bv)

    iota = lax.iota(jnp.int32, L)
    ibs = [iota * K + v * L * K for v in range(D // L)]

    def chunk_body(c, carry):
        e0 = base + c * CH * D
        pltpu.sync_copy(x_hbm.at[pl.ds(e0, CH * D)], xin)

        def row_body(r, rc):
            off = r * D
            nv = D // L
            xvs = [xin[pl.ds(off + v * L, L)] for v in range(nv)]
            labs = list(ibs)
            for s in (32, 16, 8, 4, 2, 1):
                gs = [plsc.load_gather(xkv, [labs[v] + (s - 1)])
                      for v in range(nv)]
                labs = [jnp.where(gs[v] < xvs[v], labs[v] + s, labs[v])
                        for v in range(nv)]
            for v in range(nv):
                ia = jnp.minimum(jnp.maximum(labs[v] - 1, ibs[v]),
                                 ibs[v] + (K - 2))
                a = plsc.load_gather(av, [ia])
                b = plsc.load_gather(bv, [ia])
                yout[pl.ds(off + v * L, L)] = a * xvs[v] + b
            return rc

        lax.fori_loop(0, CH, row_body, 0)
        pltpu.sync_copy(yout, out_hbm.at[pl.ds(e0, CH * D)])
        return carry

    lax.fori_loop(0, nchunk, chunk_body, 0)


def kernel(x, xk, delta_raw, scale_raw, shift):
    f32 = jnp.float32
    n = x.shape[0]
    xkT = xk.T.astype(f32)
    dpT = jnp.pad(delta_raw, ((0, 0), (0, 1))).T.astype(f32)
    sc = scale_raw[None, :].astype(f32)
    sh = shift[None, :].astype(f32)

    aT, bT = pl.pallas_call(
        _prep_kernel,
        out_shape=[jax.ShapeDtypeStruct((K, D), f32)] * 2,
    )(xkT, dpT, sc, sh)

    xkf = xk.astype(f32).reshape(-1)
    af = aT.T.reshape(-1)
    bf = bT.T.reshape(-1)
    xf = x.reshape(-1)

    mesh = plsc.VectorSubcoreMesh(core_axis_name="c", subcore_axis_name="s")
    run = functools.partial(
        pl.kernel,
        mesh=mesh,
        compiler_params=pltpu.CompilerParams(needs_layout_passes=False),
        out_type=jax.ShapeDtypeStruct((n * D,), f32),
        scratch_types=[
            pltpu.VMEM((D * K,), f32),
            pltpu.VMEM((D * K,), f32),
            pltpu.VMEM((D * K,), f32),
            pltpu.VMEM((CH * D,), f32),
            pltpu.VMEM((CH * D,), f32),
        ],
    )(_sc_spline)
    out = run(xf, xkf, af, bf)
    return out.reshape(n, D)
